# baseline (speedup 1.0000x reference)
import jax
import jax.numpy as jnp
from jax.experimental import pallas as pl
from jax.experimental.pallas import tpu as pltpu

B, S, IN = 8, 96, 128
H, M, N = 2, 32, 512
STATE, OUT, NS = 256, 126, 3
EPS = 1e-12
CIN = IN + H * M + STATE
PHEAD = NS + 1 + 3 + 1 + M + M + M + 1 + 1
TOT = STATE + OUT + H * PHEAD
HB = H * B
BM = B * M
f32 = jnp.float32
bf16 = jnp.bfloat16


def _roll_lanes(x, k):
    return jnp.concatenate([x[:, k:], x[:, :k]], axis=-1)


def _dwm_kernel(x_ref, w_ref, b_ref, out_ref, mem_ref, membf_ref):
    a0 = (jax.lax.broadcasted_iota(jnp.int32, (HB, N), 1) == 0).astype(f32)
    row = jax.lax.broadcasted_iota(jnp.int32, (HB, BM), 0) % B
    col = jax.lax.broadcasted_iota(jnp.int32, (HB, BM), 1) // M
    rmask = (row == col).astype(f32)
    rmask_bf = rmask.astype(bf16)
    mem_ref[...] = jnp.full((B, M, N), 0.01, f32)
    membf_ref[...] = jnp.full((BM, N), 0.01, bf16)

    def body(t, carry):
        state, wt_all, wd_all = carry
        x_t = x_ref[pl.ds(t, 1)].reshape(B, IN)
        o = jax.lax.dot_general(
            wt_all.astype(bf16), membf_ref[...],
            ((( 1,), (1,)), ((), ())), preferred_element_type=f32) * rmask
        o = o + _roll_lanes(o, 128)
        o = o + _roll_lanes(o, 64)
        rr = (o + _roll_lanes(o, 32))[:, :M]
        comb = jnp.concatenate([x_t, rr[:B], rr[B:], state], axis=-1)
        res = jnp.dot(comb.astype(bf16), w_ref[...],
                      preferred_element_type=f32) + b_ref[...]
        state_n = jax.nn.sigmoid(res[:, :STATE])
        out_ref[pl.ds(t, 1)] = res[:, STATE:STATE + OUT].reshape(1, B, OUT)

        P = STATE + OUT
        r0, r1 = res[:, P:P + PHEAD], res[:, P + PHEAD:P + 2 * PHEAD]
        rh = jnp.concatenate([r0, r1], axis=0)
        s_all = jax.nn.softmax(jax.nn.softplus(rh[:, 0:3]), axis=-1)
        jd_all = jax.nn.sigmoid(rh[:, 3:4])
        j_all = jax.nn.softmax(rh[:, 4:7], axis=-1)
        gamma_all = 1.0 + jax.nn.softplus(rh[:, 7:8])
        erase_all = jax.nn.sigmoid(rh[:, 8:8 + M])
        add_all = rh[:, 8 + M:8 + 2 * M]
        k_all = jnp.tanh(rh[:, 8 + 2 * M:8 + 3 * M])
        beta_all = jax.nn.softplus(rh[:, 104:105])
        g_all = jax.nn.sigmoid(rh[:, 105:106])

        mem = mem_ref[...]
        wt0, wt1 = wt_all[:B], wt_all[B:]
        fac = (1.0 - erase_all[:B][:, :, None] * wt0[:, None, :]) \
            * (1.0 - erase_all[B:][:, :, None] * wt1[:, None, :])
        mem = mem * fac \
            + add_all[:B][:, :, None] * wt0[:, None, :] \
            + add_all[B:][:, :, None] * wt1[:, None, :]
        mem_ref[...] = mem
        mem_bf = mem.astype(bf16).reshape(BM, N)
        membf_ref[...] = mem_bf

        denom = jnp.sqrt(jnp.sum(mem * mem, axis=1)) + EPS
        denom2 = jnp.concatenate([denom, denom], axis=0)
        kn = k_all / (jnp.sqrt(jnp.sum(k_all * k_all, axis=-1,
                                       keepdims=True)) + EPS)
        kn_bd = pltpu.repeat(kn.astype(bf16), B, axis=1) * rmask_bf
        wtk = jax.lax.dot_general(
            kn_bd, mem_bf,
            (((1,), (0,)), ((), ())), preferred_element_type=f32) / denom2
        wtc = g_all * jax.nn.softmax(beta_all * wtk, axis=-1) \
            + (1.0 - g_all) * wt_all
        conv = s_all[:, 0:1] * _roll_lanes(wtc, 1) \
            + s_all[:, 1:2] * wtc \
            + s_all[:, 2:3] * _roll_lanes(wtc, N - 1)
        wt_sh = jnp.exp2(gamma_all * jnp.log2(conv + EPS))
        wtn = wt_sh / jnp.sum(wt_sh, axis=-1, keepdims=True)
        wd_n = (1.0 - jd_all) * wd_all + jd_all * wtn
        wt_f = j_all[:, 0:1] * wtn + j_all[:, 1:2] * a0 + j_all[:, 2:3] * wd_all
        return (state_n, wt_f, wd_n)

    init = (jnp.ones((B, STATE), f32), a0, a0)
    jax.lax.fori_loop(0, S, body, init)


def kernel(x, W_s, b_s, W_o, b_o, W_u, b_u):
    w_cat = jnp.concatenate([W_s, W_o, W_u], axis=1).astype(bf16)
    b_cat = jnp.concatenate([b_s, b_o, b_u]).reshape(1, TOT)
    xT = jnp.swapaxes(x, 0, 1)
    out = pl.pallas_call(
        _dwm_kernel,
        grid=(1,),
        in_specs=[
            pl.BlockSpec((S, B, IN), lambda i: (0, 0, 0)),
            pl.BlockSpec((CIN, TOT), lambda i: (0, 0)),
            pl.BlockSpec((1, TOT), lambda i: (0, 0)),
        ],
        out_specs=pl.BlockSpec((S, B, OUT), lambda i: (0, 0, 0)),
        out_shape=jax.ShapeDtypeStruct((S, B, OUT), f32),
        scratch_shapes=[pltpu.VMEM((B, M, N), f32),
                        pltpu.VMEM((BM, N), bf16)],
        compiler_params=pltpu.CompilerParams(
            dimension_semantics=("arbitrary",),
        ),
    )(xT, w_cat, b_cat)
    return jnp.swapaxes(out, 0, 1)


# speedup vs baseline: 1.0223x; 1.0223x over previous
import jax
import jax.numpy as jnp
from jax.experimental import pallas as pl
from jax.experimental.pallas import tpu as pltpu

B, S, IN = 8, 96, 128
H, M, N = 2, 32, 512
STATE, OUT, NS = 256, 126, 3
EPS = 1e-12
CIN = IN + H * M + STATE
PHEAD = NS + 1 + 3 + 1 + M + M + M + 1 + 1
TOT = STATE + OUT + H * PHEAD
HB = H * B
BM = B * M
f32 = jnp.float32
bf16 = jnp.bfloat16


def _roll_lanes(x, k):
    return jnp.concatenate([x[:, k:], x[:, :k]], axis=-1)


def _dwm_kernel(x_ref, w_ref, b_ref, out_ref, mem_ref, membf_ref):
    a0 = (jax.lax.broadcasted_iota(jnp.int32, (HB, N), 1) == 0).astype(f32)
    row = jax.lax.broadcasted_iota(jnp.int32, (HB, BM), 0) % B
    col = jax.lax.broadcasted_iota(jnp.int32, (HB, BM), 1) // M
    rmask = (row == col).astype(f32)
    rmask_bf = rmask.astype(bf16)
    mem_ref[...] = jnp.full((B, M, N), 0.01, f32)
    membf_ref[...] = jnp.full((BM, N), 0.01, bf16)

    def step(t, carry):
        state, wt_all, wd_all = carry
        x_t = x_ref[pl.ds(t, 1)].reshape(B, IN)
        o = jax.lax.dot_general(
            wt_all.astype(bf16), membf_ref[...],
            ((( 1,), (1,)), ((), ())), preferred_element_type=f32) * rmask
        o = o + _roll_lanes(o, 128)
        o = o + _roll_lanes(o, 64)
        rr = (o + _roll_lanes(o, 32))[:, :M]
        comb = jnp.concatenate([x_t, rr[:B], rr[B:], state], axis=-1)
        res = jnp.dot(comb.astype(bf16), w_ref[...],
                      preferred_element_type=f32) + b_ref[...]
        state_n = jax.nn.sigmoid(res[:, :STATE])
        out_ref[pl.ds(t, 1)] = res[:, STATE:STATE + OUT].reshape(1, B, OUT)

        P = STATE + OUT
        r0, r1 = res[:, P:P + PHEAD], res[:, P + PHEAD:P + 2 * PHEAD]
        rh = jnp.concatenate([r0, r1], axis=0)
        s_all = jax.nn.softmax(jax.nn.softplus(rh[:, 0:3]), axis=-1)
        jd_all = jax.nn.sigmoid(rh[:, 3:4])
        j_all = jax.nn.softmax(rh[:, 4:7], axis=-1)
        gamma_all = 1.0 + jax.nn.softplus(rh[:, 7:8])
        erase_all = jax.nn.sigmoid(rh[:, 8:8 + M])
        add_all = rh[:, 8 + M:8 + 2 * M]
        k_all = jnp.tanh(rh[:, 8 + 2 * M:8 + 3 * M])
        beta_all = jax.nn.softplus(rh[:, 104:105])
        g_all = jax.nn.sigmoid(rh[:, 105:106])

        mem = mem_ref[...]
        wt0, wt1 = wt_all[:B], wt_all[B:]
        fac = (1.0 - erase_all[:B][:, :, None] * wt0[:, None, :]) \
            * (1.0 - erase_all[B:][:, :, None] * wt1[:, None, :])
        mem = mem * fac \
            + add_all[:B][:, :, None] * wt0[:, None, :] \
            + add_all[B:][:, :, None] * wt1[:, None, :]
        mem_ref[...] = mem
        mem_bf = mem.astype(bf16).reshape(BM, N)
        membf_ref[...] = mem_bf

        denom = jnp.sqrt(jnp.sum(mem * mem, axis=1)) + EPS
        denom2 = jnp.concatenate([denom, denom], axis=0)
        kn = k_all / (jnp.sqrt(jnp.sum(k_all * k_all, axis=-1,
                                       keepdims=True)) + EPS)
        kn_bd = pltpu.repeat(kn.astype(bf16), B, axis=1) * rmask_bf
        wtk = jax.lax.dot_general(
            kn_bd, mem_bf,
            (((1,), (0,)), ((), ())), preferred_element_type=f32) / denom2
        wtc = g_all * jax.nn.softmax(beta_all * wtk, axis=-1) \
            + (1.0 - g_all) * wt_all
        conv = s_all[:, 0:1] * _roll_lanes(wtc, 1) \
            + s_all[:, 1:2] * wtc \
            + s_all[:, 2:3] * _roll_lanes(wtc, N - 1)
        wt_sh = jnp.exp2(gamma_all * jnp.log2(conv + EPS))
        wtn = wt_sh / jnp.sum(wt_sh, axis=-1, keepdims=True)
        wd_n = (1.0 - jd_all) * wd_all + jd_all * wtn
        wt_f = j_all[:, 0:1] * wtn + j_all[:, 1:2] * a0 + j_all[:, 2:3] * wd_all
        return (state_n, wt_f, wd_n)

    UNROLL = 2

    def body(i, carry):
        t = i * UNROLL
        for u in range(UNROLL):
            carry = step(t + u, carry)
        return carry

    init = (jnp.ones((B, STATE), f32), a0, a0)
    jax.lax.fori_loop(0, S // UNROLL, body, init)


def kernel(x, W_s, b_s, W_o, b_o, W_u, b_u):
    w_cat = jnp.concatenate([W_s, W_o, W_u], axis=1).astype(bf16)
    b_cat = jnp.concatenate([b_s, b_o, b_u]).reshape(1, TOT)
    xT = jnp.swapaxes(x, 0, 1)
    out = pl.pallas_call(
        _dwm_kernel,
        grid=(1,),
        in_specs=[
            pl.BlockSpec((S, B, IN), lambda i: (0, 0, 0)),
            pl.BlockSpec((CIN, TOT), lambda i: (0, 0)),
            pl.BlockSpec((1, TOT), lambda i: (0, 0)),
        ],
        out_specs=pl.BlockSpec((S, B, OUT), lambda i: (0, 0, 0)),
        out_shape=jax.ShapeDtypeStruct((S, B, OUT), f32),
        scratch_shapes=[pltpu.VMEM((B, M, N), f32),
                        pltpu.VMEM((BM, N), bf16)],
        compiler_params=pltpu.CompilerParams(
            dimension_semantics=("arbitrary",),
        ),
    )(xT, w_cat, b_cat)
    return jnp.swapaxes(out, 0, 1)


# speedup vs baseline: 1.1579x; 1.1326x over previous
import jax
import jax.numpy as jnp
from jax.experimental import pallas as pl
from jax.experimental.pallas import tpu as pltpu

B, S, IN = 8, 96, 128
H, M, N = 2, 32, 512
STATE, OUT, NS = 256, 126, 3
EPS = 1e-12
CIN = IN + H * M + STATE
PHEAD = NS + 1 + 3 + 1 + M + M + M + 1 + 1
TOT = STATE + OUT + H * PHEAD
HB = H * B
f32 = jnp.float32
bf16 = jnp.bfloat16


def _roll_lanes(x, k):
    return jnp.concatenate([x[:, k:], x[:, :k]], axis=-1)


def _sum_m(x):
    p = jnp.sum(x.reshape(B, M // 8, 8, N), axis=1)
    return jnp.sum(p, axis=1)


def _dwm_kernel(x_ref, w_ref, b_ref, out_ref, mem_ref):
    a0 = (jax.lax.broadcasted_iota(jnp.int32, (HB, N), 1) == 0).astype(f32)
    mem_ref[...] = jnp.full((B, M, N), 0.01, f32)

    def step(t, carry):
        state, wt_all, wd_all = carry
        mem = mem_ref[...]
        x_t = x_ref[pl.ds(t, 1)].reshape(B, IN)
        wt0, wt1 = wt_all[:B], wt_all[B:]
        read0 = jnp.sum(wt0[:, None, :] * mem, axis=-1)
        read1 = jnp.sum(wt1[:, None, :] * mem, axis=-1)
        comb = jnp.concatenate([x_t, read0, read1, state], axis=-1)
        res = jnp.dot(comb.astype(bf16), w_ref[...],
                      preferred_element_type=f32) + b_ref[...]
        state_n = jax.nn.sigmoid(res[:, :STATE])
        out_ref[pl.ds(t, 1)] = res[:, STATE:STATE + OUT].reshape(1, B, OUT)

        P = STATE + OUT
        rh = jnp.concatenate(
            [res[:, P:P + PHEAD], res[:, P + PHEAD:P + 2 * PHEAD]], axis=0)
        s_all = jax.nn.softmax(jax.nn.softplus(rh[:, 0:3]), axis=-1)
        jd_all = jax.nn.sigmoid(rh[:, 3:4])
        j_all = jax.nn.softmax(rh[:, 4:7], axis=-1)
        gamma_all = 1.0 + jax.nn.softplus(rh[:, 7:8])
        erase_all = jax.nn.sigmoid(rh[:, 8:8 + M])
        add_all = rh[:, 8 + M:8 + 2 * M]
        k_all = jnp.tanh(rh[:, 8 + 2 * M:8 + 3 * M])
        beta_all = jax.nn.softplus(rh[:, 104:105])
        g_all = jax.nn.sigmoid(rh[:, 105:106])

        fac = (1.0 - erase_all[:B][:, :, None] * wt0[:, None, :]) \
            * (1.0 - erase_all[B:][:, :, None] * wt1[:, None, :])
        mem = mem * fac \
            + add_all[:B][:, :, None] * wt0[:, None, :] \
            + add_all[B:][:, :, None] * wt1[:, None, :]
        mem_ref[...] = mem

        denom = jnp.sqrt(_sum_m(mem * mem)) + EPS
        denom2 = jnp.concatenate([denom, denom], axis=0)
        kn = k_all / (jnp.sqrt(jnp.sum(k_all * k_all, axis=-1,
                                       keepdims=True)) + EPS)
        wtk0 = _sum_m(kn[:B][:, :, None] * mem)
        wtk1 = _sum_m(kn[B:][:, :, None] * mem)
        wtk = jnp.concatenate([wtk0, wtk1], axis=0) / denom2
        wtc = g_all * jax.nn.softmax(beta_all * wtk, axis=-1) \
            + (1.0 - g_all) * wt_all
        conv = s_all[:, 0:1] * _roll_lanes(wtc, 1) \
            + s_all[:, 1:2] * wtc \
            + s_all[:, 2:3] * _roll_lanes(wtc, N - 1)
        wt_sh = jnp.exp2(gamma_all * jnp.log2(conv + EPS))
        wtn = wt_sh / jnp.sum(wt_sh, axis=-1, keepdims=True)
        wd_n = (1.0 - jd_all) * wd_all + jd_all * wtn
        wt_f = j_all[:, 0:1] * wtn + j_all[:, 1:2] * a0 + j_all[:, 2:3] * wd_all
        return (state_n, wt_f, wd_n)

    UNROLL = 2

    def body(i, carry):
        t = i * UNROLL
        for u in range(UNROLL):
            carry = step(t + u, carry)
        return carry

    init = (jnp.ones((B, STATE), f32), a0, a0)
    jax.lax.fori_loop(0, S // UNROLL, body, init)


def kernel(x, W_s, b_s, W_o, b_o, W_u, b_u):
    w_cat = jnp.concatenate([W_s, W_o, W_u], axis=1).astype(bf16)
    b_cat = jnp.concatenate([b_s, b_o, b_u]).reshape(1, TOT)
    xT = jnp.swapaxes(x, 0, 1)
    out = pl.pallas_call(
        _dwm_kernel,
        grid=(1,),
        in_specs=[
            pl.BlockSpec((S, B, IN), lambda i: (0, 0, 0)),
            pl.BlockSpec((CIN, TOT), lambda i: (0, 0)),
            pl.BlockSpec((1, TOT), lambda i: (0, 0)),
        ],
        out_specs=pl.BlockSpec((S, B, OUT), lambda i: (0, 0, 0)),
        out_shape=jax.ShapeDtypeStruct((S, B, OUT), f32),
        scratch_shapes=[pltpu.VMEM((B, M, N), f32)],
        compiler_params=pltpu.CompilerParams(
            dimension_semantics=("arbitrary",),
        ),
    )(xT, w_cat, b_cat)
    return jnp.swapaxes(out, 0, 1)


# speedup vs baseline: 1.1733x; 1.0133x over previous
import jax
import jax.numpy as jnp
from jax.experimental import pallas as pl
from jax.experimental.pallas import tpu as pltpu

B, S, IN = 8, 96, 128
H, M, N = 2, 32, 512
STATE, OUT, NS = 256, 126, 3
EPS = 1e-12
CIN = IN + H * M + STATE
PHEAD = NS + 1 + 3 + 1 + M + M + M + 1 + 1
TOT = STATE + OUT + H * PHEAD
HB = H * B
f32 = jnp.float32
bf16 = jnp.bfloat16


def _roll_lanes(x, k):
    return jnp.concatenate([x[:, k:], x[:, :k]], axis=-1)


def _sum_m(x):
    p = jnp.sum(x.reshape(B, M // 8, 8, N), axis=1)
    return jnp.sum(p, axis=1)


def _dwm_kernel(x_ref, w_ref, b_ref, out_ref, mem_ref):
    a0 = (jax.lax.broadcasted_iota(jnp.int32, (HB, N), 1) == 0).astype(f32)
    mem_ref[...] = jnp.full((B, M, N), 0.01, f32)

    def step(t, carry):
        state, wt_all, wd_all = carry
        mem = mem_ref[...]
        x_t = x_ref[pl.ds(t, 1)].reshape(B, IN)
        wt0, wt1 = wt_all[:B], wt_all[B:]
        read0 = jnp.sum(wt0[:, None, :] * mem, axis=-1)
        read1 = jnp.sum(wt1[:, None, :] * mem, axis=-1)
        comb = jnp.concatenate([x_t, read0, read1, state], axis=-1)
        res = jnp.dot(comb.astype(bf16), w_ref[...],
                      preferred_element_type=f32) + b_ref[...]
        state_n = jax.nn.sigmoid(res[:, :STATE])
        out_ref[pl.ds(t, 1)] = res[:, STATE:STATE + OUT].reshape(1, B, OUT)

        P = STATE + OUT
        rh = jnp.concatenate(
            [res[:, P:P + PHEAD], res[:, P + PHEAD:P + 2 * PHEAD]], axis=0)
        s_all = jax.nn.softmax(jax.nn.softplus(rh[:, 0:3]), axis=-1)
        jd_all = jax.nn.sigmoid(rh[:, 3:4])
        j_all = jax.nn.softmax(rh[:, 4:7], axis=-1)
        gamma_all = 1.0 + jax.nn.softplus(rh[:, 7:8])
        erase_all = jax.nn.sigmoid(rh[:, 8:8 + M])
        add_all = rh[:, 8 + M:8 + 2 * M]
        k_all = jnp.tanh(rh[:, 8 + 2 * M:8 + 3 * M])
        beta_all = jax.nn.softplus(rh[:, 104:105])
        g_all = jax.nn.sigmoid(rh[:, 105:106])

        fac = (1.0 - erase_all[:B][:, :, None] * wt0[:, None, :]) \
            * (1.0 - erase_all[B:][:, :, None] * wt1[:, None, :])
        mem = mem * fac \
            + add_all[:B][:, :, None] * wt0[:, None, :] \
            + add_all[B:][:, :, None] * wt1[:, None, :]
        mem_ref[...] = mem

        denom = jnp.sqrt(_sum_m(mem * mem)) + EPS
        denom2 = jnp.concatenate([denom, denom], axis=0)
        kn = k_all / (jnp.sqrt(jnp.sum(k_all * k_all, axis=-1,
                                       keepdims=True)) + EPS)
        wtk0 = _sum_m(kn[:B][:, :, None] * mem)
        wtk1 = _sum_m(kn[B:][:, :, None] * mem)
        wtk = jnp.concatenate([wtk0, wtk1], axis=0) / denom2
        wtc = g_all * jax.nn.softmax(beta_all * wtk, axis=-1) \
            + (1.0 - g_all) * wt_all
        conv = s_all[:, 0:1] * _roll_lanes(wtc, 1) \
            + s_all[:, 1:2] * wtc \
            + s_all[:, 2:3] * _roll_lanes(wtc, N - 1)
        wt_sh = jnp.exp2(gamma_all * jnp.log2(conv + EPS))
        wtn = wt_sh / jnp.sum(wt_sh, axis=-1, keepdims=True)
        wd_n = (1.0 - jd_all) * wd_all + jd_all * wtn
        wt_f = j_all[:, 0:1] * wtn + j_all[:, 1:2] * a0 + j_all[:, 2:3] * wd_all
        return (state_n, wt_f, wd_n)

    UNROLL = 1

    def body(i, carry):
        t = i * UNROLL
        for u in range(UNROLL):
            carry = step(t + u, carry)
        return carry

    init = (jnp.ones((B, STATE), f32), a0, a0)
    jax.lax.fori_loop(0, S // UNROLL, body, init)


def kernel(x, W_s, b_s, W_o, b_o, W_u, b_u):
    w_cat = jnp.concatenate([W_s, W_o, W_u], axis=1).astype(bf16)
    b_cat = jnp.concatenate([b_s, b_o, b_u]).reshape(1, TOT)
    xT = jnp.swapaxes(x, 0, 1)
    out = pl.pallas_call(
        _dwm_kernel,
        grid=(1,),
        in_specs=[
            pl.BlockSpec((S, B, IN), lambda i: (0, 0, 0)),
            pl.BlockSpec((CIN, TOT), lambda i: (0, 0)),
            pl.BlockSpec((1, TOT), lambda i: (0, 0)),
        ],
        out_specs=pl.BlockSpec((S, B, OUT), lambda i: (0, 0, 0)),
        out_shape=jax.ShapeDtypeStruct((S, B, OUT), f32),
        scratch_shapes=[pltpu.VMEM((B, M, N), f32)],
        compiler_params=pltpu.CompilerParams(
            dimension_semantics=("arbitrary",),
        ),
    )(xT, w_cat, b_cat)
    return jnp.swapaxes(out, 0, 1)


# speedup vs baseline: 1.1734x; 1.0000x over previous
import jax
import jax.numpy as jnp
from jax.experimental import pallas as pl
from jax.experimental.pallas import tpu as pltpu

B, S, IN = 8, 96, 128
H, M, N = 2, 32, 512
STATE, OUT, NS = 256, 126, 3
EPS = 1e-12
CIN = IN + H * M + STATE
PHEAD = NS + 1 + 3 + 1 + M + M + M + 1 + 1
TOT = STATE + OUT + H * PHEAD
HB = H * B
f32 = jnp.float32
bf16 = jnp.bfloat16


def _roll_lanes(x, k):
    return jnp.concatenate([x[:, k:], x[:, :k]], axis=-1)


def _sum_m(x):
    return jnp.sum(x, axis=1)


def _dwm_kernel(x_ref, w_ref, b_ref, out_ref, mem_ref):
    a0 = (jax.lax.broadcasted_iota(jnp.int32, (HB, N), 1) == 0).astype(f32)
    mem_ref[...] = jnp.full((B, M, N), 0.01, f32)

    def step(t, carry):
        state, wt_all, wd_all = carry
        mem = mem_ref[...]
        x_t = x_ref[pl.ds(t, 1)].reshape(B, IN)
        wt0, wt1 = wt_all[:B], wt_all[B:]
        read0 = jnp.sum(wt0[:, None, :] * mem, axis=-1)
        read1 = jnp.sum(wt1[:, None, :] * mem, axis=-1)
        comb = jnp.concatenate([x_t, read0, read1, state], axis=-1)
        res = jnp.dot(comb.astype(bf16), w_ref[...],
                      preferred_element_type=f32) + b_ref[...]
        state_n = jax.nn.sigmoid(res[:, :STATE])
        out_ref[pl.ds(t, 1)] = res[:, STATE:STATE + OUT].reshape(1, B, OUT)

        P = STATE + OUT
        rh = jnp.concatenate(
            [res[:, P:P + PHEAD], res[:, P + PHEAD:P + 2 * PHEAD]], axis=0)
        s_all = jax.nn.softmax(jax.nn.softplus(rh[:, 0:3]), axis=-1)
        jd_all = jax.nn.sigmoid(rh[:, 3:4])
        j_all = jax.nn.softmax(rh[:, 4:7], axis=-1)
        gamma_all = 1.0 + jax.nn.softplus(rh[:, 7:8])
        erase_all = jax.nn.sigmoid(rh[:, 8:8 + M])
        add_all = rh[:, 8 + M:8 + 2 * M]
        k_all = jnp.tanh(rh[:, 8 + 2 * M:8 + 3 * M])
        beta_all = jax.nn.softplus(rh[:, 104:105])
        g_all = jax.nn.sigmoid(rh[:, 105:106])

        fac = (1.0 - erase_all[:B][:, :, None] * wt0[:, None, :]) \
            * (1.0 - erase_all[B:][:, :, None] * wt1[:, None, :])
        mem = mem * fac \
            + add_all[:B][:, :, None] * wt0[:, None, :] \
            + add_all[B:][:, :, None] * wt1[:, None, :]
        mem_ref[...] = mem

        denom = jnp.sqrt(_sum_m(mem * mem)) + EPS
        denom2 = jnp.concatenate([denom, denom], axis=0)
        kn = k_all / (jnp.sqrt(jnp.sum(k_all * k_all, axis=-1,
                                       keepdims=True)) + EPS)
        wtk0 = _sum_m(kn[:B][:, :, None] * mem)
        wtk1 = _sum_m(kn[B:][:, :, None] * mem)
        wtk = jnp.concatenate([wtk0, wtk1], axis=0) / denom2
        wtc = g_all * jax.nn.softmax(beta_all * wtk, axis=-1) \
            + (1.0 - g_all) * wt_all
        conv = s_all[:, 0:1] * _roll_lanes(wtc, 1) \
            + s_all[:, 1:2] * wtc \
            + s_all[:, 2:3] * _roll_lanes(wtc, N - 1)
        wt_sh = jnp.exp2(gamma_all * jnp.log2(conv + EPS))
        wtn = wt_sh / jnp.sum(wt_sh, axis=-1, keepdims=True)
        wd_n = (1.0 - jd_all) * wd_all + jd_all * wtn
        wt_f = j_all[:, 0:1] * wtn + j_all[:, 1:2] * a0 + j_all[:, 2:3] * wd_all
        return (state_n, wt_f, wd_n)

    UNROLL = 1

    def body(i, carry):
        t = i * UNROLL
        for u in range(UNROLL):
            carry = step(t + u, carry)
        return carry

    init = (jnp.ones((B, STATE), f32), a0, a0)
    jax.lax.fori_loop(0, S // UNROLL, body, init)


def kernel(x, W_s, b_s, W_o, b_o, W_u, b_u):
    w_cat = jnp.concatenate([W_s, W_o, W_u], axis=1).astype(bf16)
    b_cat = jnp.concatenate([b_s, b_o, b_u]).reshape(1, TOT)
    xT = jnp.swapaxes(x, 0, 1)
    out = pl.pallas_call(
        _dwm_kernel,
        grid=(1,),
        in_specs=[
            pl.BlockSpec((S, B, IN), lambda i: (0, 0, 0)),
            pl.BlockSpec((CIN, TOT), lambda i: (0, 0)),
            pl.BlockSpec((1, TOT), lambda i: (0, 0)),
        ],
        out_specs=pl.BlockSpec((S, B, OUT), lambda i: (0, 0, 0)),
        out_shape=jax.ShapeDtypeStruct((S, B, OUT), f32),
        scratch_shapes=[pltpu.VMEM((B, M, N), f32)],
        compiler_params=pltpu.CompilerParams(
            dimension_semantics=("arbitrary",),
        ),
    )(xT, w_cat, b_cat)
    return jnp.swapaxes(out, 0, 1)


# speedup vs baseline: 1.2978x; 1.1060x over previous
import jax
import jax.numpy as jnp
from jax.experimental import pallas as pl
from jax.experimental.pallas import tpu as pltpu

B, S, IN = 8, 96, 128
H, M, N = 2, 32, 512
STATE, OUT, NS = 256, 126, 3
EPS = 1e-12
CIN = IN + H * M + STATE
PHEAD = NS + 1 + 3 + 1 + M + M + M + 1 + 1
TOT = STATE + OUT + H * PHEAD
f32 = jnp.float32
bf16 = jnp.bfloat16


def _roll_m1(x):
    return jnp.concatenate([x[:, 1:], x[:, :1]], axis=-1)


def _roll_p1(x):
    return jnp.concatenate([x[:, -1:], x[:, :-1]], axis=-1)


def _dwm_kernel(x_ref, w_ref, b_ref, out_ref, mem_ref):
    a0 = (jax.lax.broadcasted_iota(jnp.int32, (B, N), 1) == 0).astype(f32)
    mem_ref[...] = jnp.full((B, M, N), 0.01, f32)

    def step(t, carry):
        state, wt0, wt1, wd0, wd1 = carry
        mem = mem_ref[...]
        x_t = x_ref[pl.ds(t, 1)].reshape(B, IN)
        read0 = jnp.sum(wt0[:, None, :] * mem, axis=-1)
        read1 = jnp.sum(wt1[:, None, :] * mem, axis=-1)
        comb = jnp.concatenate([x_t, read0, read1, state], axis=-1)
        res = jnp.dot(comb.astype(bf16), w_ref[...],
                      preferred_element_type=f32) + b_ref[...]
        state_n = jax.nn.sigmoid(res[:, :STATE])
        out_ref[pl.ds(t, 1)] = res[:, STATE:STATE + OUT].reshape(1, B, OUT)

        P = STATE + OUT
        pr = []
        for h in range(H):
            r = res[:, P + h * PHEAD:P + (h + 1) * PHEAD]
            pr.append(dict(
                s=jax.nn.softmax(jax.nn.softplus(r[:, 0:3]), axis=-1),
                jd=jax.nn.sigmoid(r[:, 3:4]),
                j=jax.nn.softmax(r[:, 4:7], axis=-1),
                gamma=1.0 + jax.nn.softplus(r[:, 7:8]),
                erase=jax.nn.sigmoid(r[:, 8:8 + M]),
                add=r[:, 8 + M:8 + 2 * M],
                k=jnp.tanh(r[:, 8 + 2 * M:8 + 3 * M]),
                beta=jax.nn.softplus(r[:, 104:105]),
                g=jax.nn.sigmoid(r[:, 105:106]),
            ))

        f0 = 1.0 - pr[0]["erase"][:, :, None] * wt0[:, None, :]
        f1 = 1.0 - pr[1]["erase"][:, :, None] * wt1[:, None, :]
        mem = mem * (f0 * f1) \
            + pr[0]["add"][:, :, None] * wt0[:, None, :] \
            + pr[1]["add"][:, :, None] * wt1[:, None, :]
        mem_ref[...] = mem

        denom = jnp.sqrt(jnp.sum(mem * mem, axis=1)) + EPS
        wts_new = []
        wds_new = []
        for h, wt, wd in ((0, wt0, wd0), (1, wt1, wd1)):
            p = pr[h]
            kk = p["k"]
            kn = kk / (jnp.sqrt(jnp.sum(kk * kk, axis=-1, keepdims=True)) + EPS)
            wtk = jnp.sum(kn[:, :, None] * mem, axis=1) / denom
            wtc = p["g"] * jax.nn.softmax(p["beta"] * wtk, axis=-1) \
                + (1.0 - p["g"]) * wt
            conv = p["s"][:, 0:1] * _roll_m1(wtc) \
                + p["s"][:, 1:2] * wtc \
                + p["s"][:, 2:3] * _roll_p1(wtc)
            wt_sh = jnp.exp2(p["gamma"] * jnp.log2(conv + EPS))
            wtn = wt_sh / jnp.sum(wt_sh, axis=-1, keepdims=True)
            wd_n = (1.0 - p["jd"]) * wd + p["jd"] * wtn
            wt_f = p["j"][:, 0:1] * wtn + p["j"][:, 1:2] * a0 + p["j"][:, 2:3] * wd
            wts_new.append(wt_f)
            wds_new.append(wd_n)

        return (state_n, wts_new[0], wts_new[1], wds_new[0], wds_new[1])

    init = (jnp.ones((B, STATE), f32), a0, a0, a0, a0)
    jax.lax.fori_loop(0, S, step, init)


def kernel(x, W_s, b_s, W_o, b_o, W_u, b_u):
    w_cat = jnp.concatenate([W_s, W_o, W_u], axis=1).astype(bf16)
    b_cat = jnp.concatenate([b_s, b_o, b_u]).reshape(1, TOT)
    xT = jnp.swapaxes(x, 0, 1)
    out = pl.pallas_call(
        _dwm_kernel,
        grid=(1,),
        in_specs=[
            pl.BlockSpec((S, B, IN), lambda i: (0, 0, 0)),
            pl.BlockSpec((CIN, TOT), lambda i: (0, 0)),
            pl.BlockSpec((1, TOT), lambda i: (0, 0)),
        ],
        out_specs=pl.BlockSpec((S, B, OUT), lambda i: (0, 0, 0)),
        out_shape=jax.ShapeDtypeStruct((S, B, OUT), f32),
        scratch_shapes=[pltpu.VMEM((B, M, N), f32)],
        compiler_params=pltpu.CompilerParams(
            dimension_semantics=("arbitrary",),
        ),
    )(xT, w_cat, b_cat)
    return jnp.swapaxes(out, 0, 1)


# speedup vs baseline: 1.3682x; 1.0543x over previous
import jax
import jax.numpy as jnp
from jax.experimental import pallas as pl
from jax.experimental.pallas import tpu as pltpu

B, S, IN = 8, 96, 128
H, M, N = 2, 32, 512
STATE, OUT, NS = 256, 126, 3
EPS = 1e-12
CIN = IN + H * M + STATE
PHEAD = NS + 1 + 3 + 1 + M + M + M + 1 + 1
TOT = STATE + OUT + H * PHEAD
f32 = jnp.float32
bf16 = jnp.bfloat16


def _roll_m1(x):
    return jnp.concatenate([x[:, 1:], x[:, :1]], axis=-1)


def _roll_p1(x):
    return jnp.concatenate([x[:, -1:], x[:, :-1]], axis=-1)


def _dwm_kernel(x_ref, w_ref, b_ref, out_ref, mem_ref):
    a0 = (jax.lax.broadcasted_iota(jnp.int32, (B, N), 1) == 0).astype(f32)
    mem_ref[...] = jnp.full((B, M, N), 0.01, f32)

    def step(t, carry):
        state, wt0, wt1, wd0, wd1 = carry
        mem = mem_ref[...]
        x_t = x_ref[pl.ds(t, 1)].reshape(B, IN)
        read0 = jnp.sum(wt0[:, None, :] * mem, axis=-1)
        read1 = jnp.sum(wt1[:, None, :] * mem, axis=-1)
        comb = jnp.concatenate([x_t, read0, read1, state], axis=-1)
        res = jnp.dot(comb.astype(bf16), w_ref[...],
                      preferred_element_type=f32) + b_ref[...]
        state_n = jax.nn.sigmoid(res[:, :STATE])
        out_ref[pl.ds(t, 1)] = res[:, STATE:STATE + OUT].reshape(1, B, OUT)

        P = STATE + OUT
        pr = []
        for h in range(H):
            r = res[:, P + h * PHEAD:P + (h + 1) * PHEAD]
            pr.append(dict(
                s=jax.nn.softmax(jax.nn.softplus(r[:, 0:3]), axis=-1),
                jd=jax.nn.sigmoid(r[:, 3:4]),
                j=jax.nn.softmax(r[:, 4:7], axis=-1),
                gamma=1.0 + jax.nn.softplus(r[:, 7:8]),
                erase=jax.nn.sigmoid(r[:, 8:8 + M]),
                add=r[:, 8 + M:8 + 2 * M],
                k=jnp.tanh(r[:, 8 + 2 * M:8 + 3 * M]),
                beta=jax.nn.softplus(r[:, 104:105]),
                g=jax.nn.sigmoid(r[:, 105:106]),
            ))

        f0 = 1.0 - pr[0]["erase"][:, :, None] * wt0[:, None, :]
        f1 = 1.0 - pr[1]["erase"][:, :, None] * wt1[:, None, :]
        mem = mem * (f0 * f1) \
            + pr[0]["add"][:, :, None] * wt0[:, None, :] \
            + pr[1]["add"][:, :, None] * wt1[:, None, :]
        mem_ref[...] = mem

        denom = jnp.sqrt(jnp.sum(mem * mem, axis=1)) + EPS
        wts_new = []
        wds_new = []
        for h, wt, wd in ((0, wt0, wd0), (1, wt1, wd1)):
            p = pr[h]
            kk = p["k"]
            kn = kk / (jnp.sqrt(jnp.sum(kk * kk, axis=-1, keepdims=True)) + EPS)
            wtk = jnp.sum(kn[:, :, None] * mem, axis=1) / denom
            e = jnp.exp(p["beta"] * wtk)
            sm = e / jnp.sum(e, axis=-1, keepdims=True)
            wtc = p["g"] * sm + (1.0 - p["g"]) * wt
            conv = p["s"][:, 0:1] * _roll_m1(wtc) \
                + p["s"][:, 1:2] * wtc \
                + p["s"][:, 2:3] * _roll_p1(wtc)
            wt_sh = jnp.exp2(p["gamma"] * jnp.log2(conv + EPS))
            wtn = wt_sh / jnp.sum(wt_sh, axis=-1, keepdims=True)
            wd_n = (1.0 - p["jd"]) * wd + p["jd"] * wtn
            wt_f = p["j"][:, 0:1] * wtn + p["j"][:, 1:2] * a0 + p["j"][:, 2:3] * wd
            wts_new.append(wt_f)
            wds_new.append(wd_n)

        return (state_n, wts_new[0], wts_new[1], wds_new[0], wds_new[1])

    init = (jnp.ones((B, STATE), f32), a0, a0, a0, a0)
    jax.lax.fori_loop(0, S, step, init)


def kernel(x, W_s, b_s, W_o, b_o, W_u, b_u):
    w_cat = jnp.concatenate([W_s, W_o, W_u], axis=1).astype(bf16)
    b_cat = jnp.concatenate([b_s, b_o, b_u]).reshape(1, TOT)
    xT = jnp.swapaxes(x, 0, 1)
    out = pl.pallas_call(
        _dwm_kernel,
        grid=(1,),
        in_specs=[
            pl.BlockSpec((S, B, IN), lambda i: (0, 0, 0)),
            pl.BlockSpec((CIN, TOT), lambda i: (0, 0)),
            pl.BlockSpec((1, TOT), lambda i: (0, 0)),
        ],
        out_specs=pl.BlockSpec((S, B, OUT), lambda i: (0, 0, 0)),
        out_shape=jax.ShapeDtypeStruct((S, B, OUT), f32),
        scratch_shapes=[pltpu.VMEM((B, M, N), f32)],
        compiler_params=pltpu.CompilerParams(
            dimension_semantics=("arbitrary",),
        ),
    )(xT, w_cat, b_cat)
    return jnp.swapaxes(out, 0, 1)
